# Initial kernel scaffold; baseline (speedup 1.0000x reference)
import jax
import jax.numpy as jnp
from jax.experimental import pallas as pl


def kernel(x, edge_index, edge_attr, W1, b1, W2, b2, Wf1, bf1, Wf2, bf2):
    raise NotImplementedError("write your pallas kernel here")


# speedup vs baseline: 1.7340x; 1.7340x over previous
import functools
import math

import jax
import jax.numpy as jnp
from jax.experimental import pallas as pl
from jax.experimental.pallas import tpu as pltpu
from jax.experimental.pallas import tpu_sc as plsc

N = 10000
E = 320000
D = 128
NG = 16

NUM_CORES = 2
NUM_SUBCORES = 16
NUM_WORKERS = NUM_CORES * NUM_SUBCORES
EPW = E // NUM_WORKERS
B = 80
N_PAD = 10240
ROWS_PER_SUBCORE = N_PAD // NUM_SUBCORES
ZROWS = 80

_LOG2 = math.log(2.0)
_PREC = jax.lax.Precision.HIGHEST


def _dot(a, b):
    return jnp.dot(a, b, preferred_element_type=jnp.float32, precision=_PREC)


def _xlin_body(x_ref, w1_ref, b1_ref, o_ref):
    o_ref[...] = _dot(x_ref[...], w1_ref[...]) + b1_ref[...]


def _filter_body(ea_ref, wf1_ref, bf1_ref, wf2_ref, bf2_ref, o_ref):
    h = _dot(ea_ref[...], wf1_ref[...]) + bf1_ref[...]
    h = jnp.maximum(h, 0.0) + jnp.log1p(jnp.exp(-jnp.abs(h))) - _LOG2
    o_ref[...] = _dot(h, wf2_ref[...]) + bf2_ref[...]


def _out_body(agg_ref, w2_ref, b2_ref, o_ref):
    a = agg_ref[0] + agg_ref[1]
    o_ref[...] = _dot(a, w2_ref[...]) + b2_ref[...]


def _sc_edge_body(xlin_hbm, col_hbm, row_hbm, w_hbm, out_hbm,
                  colv, rowv, rows, wv, agg, sem_g, sem_w):
    cid = jax.lax.axis_index("core")
    sid = jax.lax.axis_index("subcore")

    @pl.loop(0, ZROWS)
    def _(i):
        for j in range(0, D, 16):
            rows[i, pl.ds(j, 16)] = jnp.zeros((16,), jnp.float32)

    zsrc = rows.at[pl.ds(0, ZROWS)]
    for k in range(ROWS_PER_SUBCORE // ZROWS):
        pltpu.sync_copy(zsrc, agg.at[pl.ds(sid * ROWS_PER_SUBCORE + k * ZROWS,
                                           ZROWS)])
    plsc.subcore_barrier()

    wid = cid * NUM_SUBCORES + sid

    @pl.loop(0, EPW, step=B)
    def _(off):
        base = wid * EPW + off
        pltpu.sync_copy(col_hbm.at[pl.ds(base, B)], colv)
        pltpu.sync_copy(row_hbm.at[pl.ds(base, B)], rowv)
        cp_w = pltpu.async_copy(w_hbm.at[pl.ds(base, B)], wv, sem_w)
        cp_g = pltpu.async_copy(xlin_hbm.at[colv], rows, sem_g)
        cp_w.wait()
        cp_g.wait()

        @pl.loop(0, B)
        def _(i):
            for j in range(0, D, 16):
                s = (i, pl.ds(j, 16))
                rows[s] = rows[s] * wv[s]

        pltpu.sync_copy(rows, agg.at[rowv], add=True)

    plsc.subcore_barrier()
    pltpu.sync_copy(agg.at[pl.ds(sid * ROWS_PER_SUBCORE, ROWS_PER_SUBCORE)],
                    out_hbm.at[cid, pl.ds(sid * ROWS_PER_SUBCORE,
                                          ROWS_PER_SUBCORE)])


@jax.jit
def kernel(x, edge_index, edge_attr, W1, b1, W2, b2, Wf1, bf1, Wf2, bf2):
    f32 = jnp.float32
    b1r = b1.reshape(1, D)
    b2r = b2.reshape(1, D)
    bf1r = bf1.reshape(1, D)
    bf2r = bf2.reshape(1, D)
    row = edge_index[0]
    col = edge_index[1]

    xlin = pl.pallas_call(
        _xlin_body,
        grid=(10,),
        in_specs=[
            pl.BlockSpec((N // 10, D), lambda i: (i, 0)),
            pl.BlockSpec((D, D), lambda i: (0, 0)),
            pl.BlockSpec((1, D), lambda i: (0, 0)),
        ],
        out_specs=pl.BlockSpec((N // 10, D), lambda i: (i, 0)),
        out_shape=jax.ShapeDtypeStruct((N, D), f32),
    )(x, W1, b1r)

    BE = 4000
    w_edge = pl.pallas_call(
        _filter_body,
        grid=(E // BE,),
        in_specs=[
            pl.BlockSpec((BE, NG), lambda i: (i, 0)),
            pl.BlockSpec((NG, D), lambda i: (0, 0)),
            pl.BlockSpec((1, D), lambda i: (0, 0)),
            pl.BlockSpec((D, D), lambda i: (0, 0)),
            pl.BlockSpec((1, D), lambda i: (0, 0)),
        ],
        out_specs=pl.BlockSpec((BE, D), lambda i: (i, 0)),
        out_shape=jax.ShapeDtypeStruct((E, D), f32),
    )(edge_attr, Wf1, bf1r, Wf2, bf2r)

    sc_mesh = plsc.VectorSubcoreMesh(core_axis_name="core",
                                     subcore_axis_name="subcore")
    agg2 = pl.kernel(
        _sc_edge_body,
        out_type=jax.ShapeDtypeStruct((NUM_CORES, N_PAD, D), f32),
        mesh=sc_mesh,
        scratch_types=[
            pltpu.VMEM((B,), jnp.int32),
            pltpu.VMEM((B,), jnp.int32),
            pltpu.VMEM((B, D), f32),
            pltpu.VMEM((B, D), f32),
            pltpu.VMEM_SHARED((N_PAD, D), f32),
            pltpu.SemaphoreType.DMA,
            pltpu.SemaphoreType.DMA,
        ],
    )(xlin, col, row, w_edge)

    out = pl.pallas_call(
        _out_body,
        grid=(10,),
        in_specs=[
            pl.BlockSpec((NUM_CORES, N // 10, D), lambda i: (0, i, 0)),
            pl.BlockSpec((D, D), lambda i: (0, 0)),
            pl.BlockSpec((1, D), lambda i: (0, 0)),
        ],
        out_specs=pl.BlockSpec((N // 10, D), lambda i: (i, 0)),
        out_shape=jax.ShapeDtypeStruct((N, D), f32),
    )(agg2, W2, b2r)
    return out


# speedup vs baseline: 2.7338x; 1.5766x over previous
import functools
import math

import jax
import jax.numpy as jnp
from jax.experimental import pallas as pl
from jax.experimental.pallas import tpu as pltpu
from jax.experimental.pallas import tpu_sc as plsc

N = 10000
E = 320000
D = 128
NG = 16

NUM_CORES = 2
NUM_SUBCORES = 16
NUM_WORKERS = NUM_CORES * NUM_SUBCORES
EPW = E // NUM_WORKERS
B = 80
N_PAD = 10240
ROWS_PER_SUBCORE = N_PAD // NUM_SUBCORES
ZROWS = 80

_LOG2 = math.log(2.0)
_PREC = jax.lax.Precision.DEFAULT


def _dot(a, b):
    return jnp.dot(a, b, preferred_element_type=jnp.float32, precision=_PREC)


def _xlin_body(x_ref, w1_ref, b1_ref, o_ref):
    o_ref[...] = _dot(x_ref[...], w1_ref[...]) + b1_ref[...]


def _filter_body(ea_ref, wf1_ref, bf1_ref, wf2_ref, bf2_ref, o_ref):
    h = _dot(ea_ref[...], wf1_ref[...]) + bf1_ref[...]
    h = jnp.maximum(h, 0.0) + jnp.log1p(jnp.exp(-jnp.abs(h))) - _LOG2
    o_ref[...] = _dot(h, wf2_ref[...]) + bf2_ref[...]


def _out_body(agg_ref, w2_ref, b2_ref, o_ref):
    a = agg_ref[0] + agg_ref[1]
    o_ref[...] = _dot(a, w2_ref[...]) + b2_ref[...]


def _sc_edge_body(xlin_hbm, col_hbm, row_hbm, w_hbm, out_hbm,
                  colv, rowv, rows, wv, agg, sem_g, sem_w):
    cid = jax.lax.axis_index("core")
    sid = jax.lax.axis_index("subcore")

    @pl.loop(0, ZROWS)
    def _(i):
        for j in range(0, D, 16):
            rows[i, pl.ds(j, 16)] = jnp.zeros((16,), jnp.float32)

    zsrc = rows.at[pl.ds(0, ZROWS)]
    for k in range(ROWS_PER_SUBCORE // ZROWS):
        pltpu.sync_copy(zsrc, agg.at[pl.ds(sid * ROWS_PER_SUBCORE + k * ZROWS,
                                           ZROWS)])
    plsc.subcore_barrier()

    wid = cid * NUM_SUBCORES + sid

    @pl.loop(0, EPW, step=B)
    def _(off):
        base = wid * EPW + off
        pltpu.sync_copy(col_hbm.at[pl.ds(base, B)], colv)
        pltpu.sync_copy(row_hbm.at[pl.ds(base, B)], rowv)
        cp_w = pltpu.async_copy(w_hbm.at[pl.ds(base, B)], wv, sem_w)
        cp_g = pltpu.async_copy(xlin_hbm.at[colv], rows, sem_g)
        cp_w.wait()
        cp_g.wait()

        @pl.loop(0, B)
        def _(i):
            for j in range(0, D, 16):
                s = (i, pl.ds(j, 16))
                rows[s] = rows[s] * wv[s]

        pltpu.sync_copy(rows, agg.at[rowv], add=True)

    plsc.subcore_barrier()
    pltpu.sync_copy(agg.at[pl.ds(sid * ROWS_PER_SUBCORE, ROWS_PER_SUBCORE)],
                    out_hbm.at[cid, pl.ds(sid * ROWS_PER_SUBCORE,
                                          ROWS_PER_SUBCORE)])


@jax.jit
def kernel(x, edge_index, edge_attr, W1, b1, W2, b2, Wf1, bf1, Wf2, bf2):
    f32 = jnp.float32
    b1r = b1.reshape(1, D)
    b2r = b2.reshape(1, D)
    bf1r = bf1.reshape(1, D)
    bf2r = bf2.reshape(1, D)
    row = edge_index[0]
    col = edge_index[1]

    xlin = pl.pallas_call(
        _xlin_body,
        grid=(10,),
        in_specs=[
            pl.BlockSpec((N // 10, D), lambda i: (i, 0)),
            pl.BlockSpec((D, D), lambda i: (0, 0)),
            pl.BlockSpec((1, D), lambda i: (0, 0)),
        ],
        out_specs=pl.BlockSpec((N // 10, D), lambda i: (i, 0)),
        out_shape=jax.ShapeDtypeStruct((N, D), f32),
    )(x, W1, b1r)

    BE = 4000
    w_edge = pl.pallas_call(
        _filter_body,
        grid=(E // BE,),
        in_specs=[
            pl.BlockSpec((BE, NG), lambda i: (i, 0)),
            pl.BlockSpec((NG, D), lambda i: (0, 0)),
            pl.BlockSpec((1, D), lambda i: (0, 0)),
            pl.BlockSpec((D, D), lambda i: (0, 0)),
            pl.BlockSpec((1, D), lambda i: (0, 0)),
        ],
        out_specs=pl.BlockSpec((BE, D), lambda i: (i, 0)),
        out_shape=jax.ShapeDtypeStruct((E, D), f32),
    )(edge_attr, Wf1, bf1r, Wf2, bf2r)

    sc_mesh = plsc.VectorSubcoreMesh(core_axis_name="core",
                                     subcore_axis_name="subcore")
    agg2 = pl.kernel(
        _sc_edge_body,
        out_type=jax.ShapeDtypeStruct((NUM_CORES, N_PAD, D), f32),
        mesh=sc_mesh,
        scratch_types=[
            pltpu.VMEM((B,), jnp.int32),
            pltpu.VMEM((B,), jnp.int32),
            pltpu.VMEM((B, D), f32),
            pltpu.VMEM((B, D), f32),
            pltpu.VMEM_SHARED((N_PAD, D), f32),
            pltpu.SemaphoreType.DMA,
            pltpu.SemaphoreType.DMA,
        ],
    )(xlin, col, row, w_edge)

    out = pl.pallas_call(
        _out_body,
        grid=(10,),
        in_specs=[
            pl.BlockSpec((NUM_CORES, N // 10, D), lambda i: (0, i, 0)),
            pl.BlockSpec((D, D), lambda i: (0, 0)),
            pl.BlockSpec((1, D), lambda i: (0, 0)),
        ],
        out_specs=pl.BlockSpec((N // 10, D), lambda i: (i, 0)),
        out_shape=jax.ShapeDtypeStruct((N, D), f32),
    )(agg2, W2, b2r)
    return out


# speedup vs baseline: 3.8067x; 1.3924x over previous
import functools
import math

import jax
import jax.numpy as jnp
from jax.experimental import pallas as pl
from jax.experimental.pallas import tpu as pltpu
from jax.experimental.pallas import tpu_sc as plsc

N = 10000
E = 320000
D = 128
NG = 16

NUM_CORES = 2
NUM_SUBCORES = 16
NUM_WORKERS = NUM_CORES * NUM_SUBCORES
EPW = E // NUM_WORKERS
B = 80
N_PAD = 10240
ROWS_PER_SUBCORE = N_PAD // NUM_SUBCORES
ZROWS = 80

_LOG2 = math.log(2.0)
_PREC = jax.lax.Precision.DEFAULT


def _dot(a, b):
    return jnp.dot(a, b, preferred_element_type=jnp.float32, precision=_PREC)


def _xlin_body(x_ref, w1_ref, b1_ref, o_ref):
    o_ref[...] = _dot(x_ref[...], w1_ref[...]) + b1_ref[...]


def _filter_body(ea_ref, wf1_ref, bf1_ref, wf2_ref, bf2_ref, o_ref):
    h = _dot(ea_ref[...], wf1_ref[...]) + bf1_ref[...]
    h = jnp.maximum(h, 0.0) + jnp.log1p(jnp.exp(-jnp.abs(h))) - _LOG2
    o_ref[...] = _dot(h, wf2_ref[...]) + bf2_ref[...]


def _out_body(agg_ref, w2_ref, b2_ref, o_ref):
    a = agg_ref[0] + agg_ref[1]
    o_ref[...] = _dot(a, w2_ref[...]) + b2_ref[...]


NITERS = EPW // B
PAIRS = (NITERS - 1) // 2


def _sc_edge_body(xlin_hbm, col_hbm, row_hbm, w_hbm, out_hbm,
                  colv0, rowv0, rows0, wv0, colv1, rowv1, rows1, wv1,
                  agg, sg0, sw0, ss0, sg1, sw1, ss1):
    cid = jax.lax.axis_index("core")
    sid = jax.lax.axis_index("subcore")

    @pl.loop(0, ZROWS)
    def _(i):
        for j in range(0, D, 16):
            rows0[i, pl.ds(j, 16)] = jnp.zeros((16,), jnp.float32)

    zsrc = rows0.at[pl.ds(0, ZROWS)]
    for k in range(ROWS_PER_SUBCORE // ZROWS):
        pltpu.sync_copy(zsrc, agg.at[pl.ds(sid * ROWS_PER_SUBCORE + k * ZROWS,
                                           ZROWS)])
    plsc.subcore_barrier()

    wid = cid * NUM_SUBCORES + sid
    e0 = wid * EPW

    def start(base, colv, rowv, rows, wv, sg, sw):
        pltpu.sync_copy(col_hbm.at[pl.ds(base, B)], colv)
        pltpu.sync_copy(row_hbm.at[pl.ds(base, B)], rowv)
        pltpu.async_copy(w_hbm.at[pl.ds(base, B)], wv, sw)
        pltpu.async_copy(xlin_hbm.at[colv], rows, sg)

    def mult_scatter(colv, rowv, rows, wv, sg, sw, ss):
        pltpu.make_async_copy(xlin_hbm.at[colv], rows, sg).wait()
        pltpu.make_async_copy(w_hbm.at[pl.ds(0, B)], wv, sw).wait()

        @pl.loop(0, B)
        def _(i):
            for j in range(0, D, 16):
                s = (i, pl.ds(j, 16))
                wv[s] = rows[s] * wv[s]

        pltpu.async_copy(wv, agg.at[rowv], ss, add=True)

    def wait_scatter(rowv, wv, ss):
        pltpu.make_async_copy(wv, agg.at[rowv], ss).wait()

    start(e0, colv0, rowv0, rows0, wv0, sg0, sw0)
    start(e0 + B, colv1, rowv1, rows1, wv1, sg1, sw1)

    @pl.loop(0, PAIRS)
    def _(p):
        a = e0 + 2 * p * B

        mult_scatter(colv0, rowv0, rows0, wv0, sg0, sw0, ss0)
        pltpu.sync_copy(col_hbm.at[pl.ds(a + 2 * B, B)], colv0)
        pltpu.async_copy(xlin_hbm.at[colv0], rows0, sg0)

        mult_scatter(colv1, rowv1, rows1, wv1, sg1, sw1, ss1)

        wait_scatter(rowv0, wv0, ss0)
        pltpu.sync_copy(row_hbm.at[pl.ds(a + 2 * B, B)], rowv0)
        pltpu.async_copy(w_hbm.at[pl.ds(a + 2 * B, B)], wv0, sw0)

        @pl.when(p < PAIRS - 1)
        def _():
            pltpu.sync_copy(col_hbm.at[pl.ds(a + 3 * B, B)], colv1)
            pltpu.async_copy(xlin_hbm.at[colv1], rows1, sg1)
            wait_scatter(rowv1, wv1, ss1)
            pltpu.sync_copy(row_hbm.at[pl.ds(a + 3 * B, B)], rowv1)
            pltpu.async_copy(w_hbm.at[pl.ds(a + 3 * B, B)], wv1, sw1)

    mult_scatter(colv0, rowv0, rows0, wv0, sg0, sw0, ss0)
    wait_scatter(rowv0, wv0, ss0)
    wait_scatter(rowv1, wv1, ss1)

    plsc.subcore_barrier()
    pltpu.sync_copy(agg.at[pl.ds(sid * ROWS_PER_SUBCORE, ROWS_PER_SUBCORE)],
                    out_hbm.at[cid, pl.ds(sid * ROWS_PER_SUBCORE,
                                          ROWS_PER_SUBCORE)])


@jax.jit
def kernel(x, edge_index, edge_attr, W1, b1, W2, b2, Wf1, bf1, Wf2, bf2):
    f32 = jnp.float32
    b1r = b1.reshape(1, D)
    b2r = b2.reshape(1, D)
    bf1r = bf1.reshape(1, D)
    bf2r = bf2.reshape(1, D)
    row = edge_index[0]
    col = edge_index[1]

    xlin = pl.pallas_call(
        _xlin_body,
        grid=(10,),
        in_specs=[
            pl.BlockSpec((N // 10, D), lambda i: (i, 0)),
            pl.BlockSpec((D, D), lambda i: (0, 0)),
            pl.BlockSpec((1, D), lambda i: (0, 0)),
        ],
        out_specs=pl.BlockSpec((N // 10, D), lambda i: (i, 0)),
        out_shape=jax.ShapeDtypeStruct((N, D), f32),
    )(x, W1, b1r)

    BE = 4000
    w_edge = pl.pallas_call(
        _filter_body,
        grid=(E // BE,),
        in_specs=[
            pl.BlockSpec((BE, NG), lambda i: (i, 0)),
            pl.BlockSpec((NG, D), lambda i: (0, 0)),
            pl.BlockSpec((1, D), lambda i: (0, 0)),
            pl.BlockSpec((D, D), lambda i: (0, 0)),
            pl.BlockSpec((1, D), lambda i: (0, 0)),
        ],
        out_specs=pl.BlockSpec((BE, D), lambda i: (i, 0)),
        out_shape=jax.ShapeDtypeStruct((E, D), f32),
    )(edge_attr, Wf1, bf1r, Wf2, bf2r)

    sc_mesh = plsc.VectorSubcoreMesh(core_axis_name="core",
                                     subcore_axis_name="subcore")
    agg2 = pl.kernel(
        _sc_edge_body,
        out_type=jax.ShapeDtypeStruct((NUM_CORES, N_PAD, D), f32),
        mesh=sc_mesh,
        scratch_types=[
            pltpu.VMEM((B,), jnp.int32),
            pltpu.VMEM((B,), jnp.int32),
            pltpu.VMEM((B, D), f32),
            pltpu.VMEM((B, D), f32),
            pltpu.VMEM((B,), jnp.int32),
            pltpu.VMEM((B,), jnp.int32),
            pltpu.VMEM((B, D), f32),
            pltpu.VMEM((B, D), f32),
            pltpu.VMEM_SHARED((N_PAD, D), f32),
            pltpu.SemaphoreType.DMA,
            pltpu.SemaphoreType.DMA,
            pltpu.SemaphoreType.DMA,
            pltpu.SemaphoreType.DMA,
            pltpu.SemaphoreType.DMA,
            pltpu.SemaphoreType.DMA,
        ],
    )(xlin, col, row, w_edge)

    out = pl.pallas_call(
        _out_body,
        grid=(10,),
        in_specs=[
            pl.BlockSpec((NUM_CORES, N // 10, D), lambda i: (0, i, 0)),
            pl.BlockSpec((D, D), lambda i: (0, 0)),
            pl.BlockSpec((1, D), lambda i: (0, 0)),
        ],
        out_specs=pl.BlockSpec((N // 10, D), lambda i: (i, 0)),
        out_shape=jax.ShapeDtypeStruct((N, D), f32),
    )(agg2, W2, b2r)
    return out


# speedup vs baseline: 3.9068x; 1.0263x over previous
import functools
import math

import jax
import jax.numpy as jnp
from jax.experimental import pallas as pl
from jax.experimental.pallas import tpu as pltpu
from jax.experimental.pallas import tpu_sc as plsc

N = 10000
E = 320000
D = 128
NG = 16

NUM_CORES = 2
NUM_SUBCORES = 16
NUM_WORKERS = NUM_CORES * NUM_SUBCORES
EPW = E // NUM_WORKERS
B = 80
N_PAD = 10240
ROWS_PER_SUBCORE = N_PAD // NUM_SUBCORES
ZROWS = 80

_LOG2 = math.log(2.0)
_PREC = jax.lax.Precision.DEFAULT


def _dot(a, b):
    return jnp.dot(a, b, preferred_element_type=jnp.float32, precision=_PREC)


def _xlin_body(x_ref, w1_ref, b1_ref, o_ref):
    o_ref[...] = _dot(x_ref[...], w1_ref[...]) + b1_ref[...]


def _filter_body(ea_ref, wf1_ref, bf1_ref, wf2_ref, bf2_ref, o_ref):
    h = _dot(ea_ref[...], wf1_ref[...]) + bf1_ref[...]
    h = jnp.maximum(h, 0.0) + jnp.log1p(jnp.exp(-jnp.abs(h))) - _LOG2
    o_ref[...] = _dot(h, wf2_ref[...]) + bf2_ref[...]


def _out_body(*refs):
    agg_refs, (w2_ref, b2_ref, o_ref) = refs[:-3], refs[-3:]
    a = agg_refs[0][0] + agg_refs[0][1]
    for r in agg_refs[1:]:
        a = a + r[0] + r[1]
    o_ref[...] = _dot(a, w2_ref[...]) + b2_ref[...]


NCHUNK = 5
EC = E // NCHUNK
EPWC = EC // NUM_WORKERS
NITERS = EPWC // B
PAIRS = (NITERS - 1) // 2


def _sc_edge_body(chunk, xlin_hbm, col_hbm, row_hbm, w_hbm, out_hbm,
                  colv0, rowv0, rows0, wv0, colv1, rowv1, rows1, wv1,
                  agg, sg0, sw0, ss0, sg1, sw1, ss1):
    cid = jax.lax.axis_index("core")
    sid = jax.lax.axis_index("subcore")

    @pl.loop(0, ZROWS)
    def _(i):
        for j in range(0, D, 16):
            rows0[i, pl.ds(j, 16)] = jnp.zeros((16,), jnp.float32)

    zsrc = rows0.at[pl.ds(0, ZROWS)]
    for k in range(ROWS_PER_SUBCORE // ZROWS):
        pltpu.sync_copy(zsrc, agg.at[pl.ds(sid * ROWS_PER_SUBCORE + k * ZROWS,
                                           ZROWS)])
    plsc.subcore_barrier()

    wid = cid * NUM_SUBCORES + sid
    e0 = chunk * EC + wid * EPWC

    w0 = wid * EPWC

    def start(base, wbase, colv, rowv, rows, wv, sg, sw):
        pltpu.sync_copy(col_hbm.at[pl.ds(base, B)], colv)
        pltpu.sync_copy(row_hbm.at[pl.ds(base, B)], rowv)
        pltpu.async_copy(w_hbm.at[pl.ds(wbase, B)], wv, sw)
        pltpu.async_copy(xlin_hbm.at[colv], rows, sg)

    def mult_scatter(colv, rowv, rows, wv, sg, sw, ss):
        pltpu.make_async_copy(xlin_hbm.at[colv], rows, sg).wait()
        pltpu.make_async_copy(w_hbm.at[pl.ds(0, B)], wv, sw).wait()

        @pl.loop(0, B)
        def _(i):
            for j in range(0, D, 16):
                s = (i, pl.ds(j, 16))
                wv[s] = rows[s] * wv[s]

        pltpu.async_copy(wv, agg.at[rowv], ss, add=True)

    def wait_scatter(rowv, wv, ss):
        pltpu.make_async_copy(wv, agg.at[rowv], ss).wait()

    start(e0, w0, colv0, rowv0, rows0, wv0, sg0, sw0)
    start(e0 + B, w0 + B, colv1, rowv1, rows1, wv1, sg1, sw1)

    @pl.loop(0, PAIRS)
    def _(p):
        a = e0 + 2 * p * B
        aw = w0 + 2 * p * B

        mult_scatter(colv0, rowv0, rows0, wv0, sg0, sw0, ss0)
        pltpu.sync_copy(col_hbm.at[pl.ds(a + 2 * B, B)], colv0)
        pltpu.async_copy(xlin_hbm.at[colv0], rows0, sg0)

        mult_scatter(colv1, rowv1, rows1, wv1, sg1, sw1, ss1)

        wait_scatter(rowv0, wv0, ss0)
        pltpu.sync_copy(row_hbm.at[pl.ds(a + 2 * B, B)], rowv0)
        pltpu.async_copy(w_hbm.at[pl.ds(aw + 2 * B, B)], wv0, sw0)

        @pl.when(p < PAIRS - 1)
        def _():
            pltpu.sync_copy(col_hbm.at[pl.ds(a + 3 * B, B)], colv1)
            pltpu.async_copy(xlin_hbm.at[colv1], rows1, sg1)
            wait_scatter(rowv1, wv1, ss1)
            pltpu.sync_copy(row_hbm.at[pl.ds(a + 3 * B, B)], rowv1)
            pltpu.async_copy(w_hbm.at[pl.ds(aw + 3 * B, B)], wv1, sw1)

    mult_scatter(colv0, rowv0, rows0, wv0, sg0, sw0, ss0)
    wait_scatter(rowv0, wv0, ss0)
    wait_scatter(rowv1, wv1, ss1)

    plsc.subcore_barrier()
    pltpu.sync_copy(agg.at[pl.ds(sid * ROWS_PER_SUBCORE, ROWS_PER_SUBCORE)],
                    out_hbm.at[cid, pl.ds(sid * ROWS_PER_SUBCORE,
                                          ROWS_PER_SUBCORE)])


@jax.jit
def kernel(x, edge_index, edge_attr, W1, b1, W2, b2, Wf1, bf1, Wf2, bf2):
    f32 = jnp.float32
    b1r = b1.reshape(1, D)
    b2r = b2.reshape(1, D)
    bf1r = bf1.reshape(1, D)
    bf2r = bf2.reshape(1, D)
    row = edge_index[0]
    col = edge_index[1]

    xlin = pl.pallas_call(
        _xlin_body,
        grid=(10,),
        in_specs=[
            pl.BlockSpec((N // 10, D), lambda i: (i, 0)),
            pl.BlockSpec((D, D), lambda i: (0, 0)),
            pl.BlockSpec((1, D), lambda i: (0, 0)),
        ],
        out_specs=pl.BlockSpec((N // 10, D), lambda i: (i, 0)),
        out_shape=jax.ShapeDtypeStruct((N, D), f32),
    )(x, W1, b1r)

    BE = 4000
    sc_mesh = plsc.VectorSubcoreMesh(core_axis_name="core",
                                     subcore_axis_name="subcore")
    partials = []
    for c in range(NCHUNK):
        w_chunk = pl.pallas_call(
            _filter_body,
            grid=(EC // BE,),
            in_specs=[
                pl.BlockSpec((BE, NG), lambda i, c=c: (c * (EC // BE) + i, 0)),
                pl.BlockSpec((NG, D), lambda i: (0, 0)),
                pl.BlockSpec((1, D), lambda i: (0, 0)),
                pl.BlockSpec((D, D), lambda i: (0, 0)),
                pl.BlockSpec((1, D), lambda i: (0, 0)),
            ],
            out_specs=pl.BlockSpec((BE, D), lambda i: (i, 0)),
            out_shape=jax.ShapeDtypeStruct((EC, D), f32),
        )(edge_attr, Wf1, bf1r, Wf2, bf2r)

        agg_c = pl.kernel(
            functools.partial(_sc_edge_body, c),
            out_type=jax.ShapeDtypeStruct((NUM_CORES, N_PAD, D), f32),
            mesh=sc_mesh,
            scratch_types=[
                pltpu.VMEM((B,), jnp.int32),
                pltpu.VMEM((B,), jnp.int32),
                pltpu.VMEM((B, D), f32),
                pltpu.VMEM((B, D), f32),
                pltpu.VMEM((B,), jnp.int32),
                pltpu.VMEM((B,), jnp.int32),
                pltpu.VMEM((B, D), f32),
                pltpu.VMEM((B, D), f32),
                pltpu.VMEM_SHARED((N_PAD, D), f32),
                pltpu.SemaphoreType.DMA,
                pltpu.SemaphoreType.DMA,
                pltpu.SemaphoreType.DMA,
                pltpu.SemaphoreType.DMA,
                pltpu.SemaphoreType.DMA,
                pltpu.SemaphoreType.DMA,
            ],
        )(xlin, col, row, w_chunk)
        partials.append(agg_c)

    out = pl.pallas_call(
        _out_body,
        grid=(10,),
        in_specs=[pl.BlockSpec((NUM_CORES, N // 10, D), lambda i: (0, i, 0))
                  for _ in range(NCHUNK)] + [
            pl.BlockSpec((D, D), lambda i: (0, 0)),
            pl.BlockSpec((1, D), lambda i: (0, 0)),
        ],
        out_specs=pl.BlockSpec((N // 10, D), lambda i: (i, 0)),
        out_shape=jax.ShapeDtypeStruct((N, D), f32),
    )(*partials, W2, b2r)
    return out


# speedup vs baseline: 3.9120x; 1.0013x over previous
import functools
import math

import jax
import jax.numpy as jnp
from jax.experimental import pallas as pl
from jax.experimental.pallas import tpu as pltpu
from jax.experimental.pallas import tpu_sc as plsc

N = 10000
E = 320000
D = 128
NG = 16

NUM_CORES = 2
NUM_SUBCORES = 16
NUM_WORKERS = NUM_CORES * NUM_SUBCORES
EPW = E // NUM_WORKERS
B = 80
N_PAD = 10240
ROWS_PER_SUBCORE = N_PAD // NUM_SUBCORES
ZROWS = 80

_LOG2 = math.log(2.0)
_PREC = jax.lax.Precision.DEFAULT


def _dot(a, b):
    return jnp.dot(a, b, preferred_element_type=jnp.float32, precision=_PREC)


def _xlin_body(x_ref, w1_ref, b1_ref, o_ref):
    o_ref[...] = _dot(x_ref[...], w1_ref[...]) + b1_ref[...]


def _filter_body(ea_ref, wf1_ref, bf1_ref, wf2_ref, bf2_ref, o_ref):
    h = _dot(ea_ref[...], wf1_ref[...]) + bf1_ref[...]
    h = jnp.maximum(h, 0.0) + jnp.log1p(jnp.exp(-jnp.abs(h))) - _LOG2
    o_ref[...] = _dot(h, wf2_ref[...]) + bf2_ref[...]


def _out_body(*refs):
    agg_refs, (w2_ref, b2_ref, o_ref) = refs[:-3], refs[-3:]
    a = agg_refs[0][0] + agg_refs[0][1]
    for r in agg_refs[1:]:
        a = a + r[0] + r[1]
    o_ref[...] = _dot(a, w2_ref[...]) + b2_ref[...]


NCHUNK = 5
EC = E // NCHUNK
EPWC = EC // NUM_WORKERS
NITERS = EPWC // B
PAIRS = (NITERS - 1) // 2


def _sc_edge_body(chunk, xlin_hbm, col_hbm, row_hbm, w_hbm, out_hbm,
                  colv0, rowv0, rows0, wv0, colv1, rowv1, rows1, wv1,
                  agg, sg0, sw0, ss0, sg1, sw1, ss1):
    cid = jax.lax.axis_index("core")
    sid = jax.lax.axis_index("subcore")

    @pl.loop(0, ZROWS)
    def _(i):
        for j in range(0, D, 16):
            rows0[i, pl.ds(j, 16)] = jnp.zeros((16,), jnp.float32)

    zsrc = rows0.at[pl.ds(0, ZROWS)]
    for k in range(ROWS_PER_SUBCORE // ZROWS):
        pltpu.sync_copy(zsrc, agg.at[pl.ds(sid * ROWS_PER_SUBCORE + k * ZROWS,
                                           ZROWS)])
    plsc.subcore_barrier()

    wid = cid * NUM_SUBCORES + sid
    e0 = chunk * EC + wid * EPWC

    w0 = wid * EPWC

    def start(base, wbase, colv, rowv, rows, wv, sg, sw):
        pltpu.sync_copy(col_hbm.at[pl.ds(base, B)], colv)
        pltpu.sync_copy(row_hbm.at[pl.ds(base, B)], rowv)
        pltpu.async_copy(w_hbm.at[pl.ds(wbase, B)], wv, sw)
        pltpu.async_copy(xlin_hbm.at[colv], rows, sg)

    def mult_scatter(colv, rowv, rows, wv, sg, sw, ss):
        pltpu.make_async_copy(xlin_hbm.at[colv], rows, sg).wait()
        pltpu.make_async_copy(w_hbm.at[pl.ds(0, B)], wv, sw).wait()

        @pl.loop(0, B)
        def _(i):
            for j in range(0, D, 16):
                s = (i, pl.ds(j, 16))
                wv[s] = rows[s] * wv[s]

        pltpu.async_copy(wv, agg.at[rowv], ss, add=True)

    def wait_scatter(rowv, wv, ss):
        pltpu.make_async_copy(wv, agg.at[rowv], ss).wait()

    start(e0, w0, colv0, rowv0, rows0, wv0, sg0, sw0)
    start(e0 + B, w0 + B, colv1, rowv1, rows1, wv1, sg1, sw1)

    @pl.loop(0, PAIRS)
    def _(p):
        a = e0 + 2 * p * B
        aw = w0 + 2 * p * B

        mult_scatter(colv0, rowv0, rows0, wv0, sg0, sw0, ss0)
        pltpu.sync_copy(col_hbm.at[pl.ds(a + 2 * B, B)], colv0)
        pltpu.async_copy(xlin_hbm.at[colv0], rows0, sg0)

        mult_scatter(colv1, rowv1, rows1, wv1, sg1, sw1, ss1)

        wait_scatter(rowv0, wv0, ss0)
        pltpu.sync_copy(row_hbm.at[pl.ds(a + 2 * B, B)], rowv0)
        pltpu.async_copy(w_hbm.at[pl.ds(aw + 2 * B, B)], wv0, sw0)

        @pl.when(p < PAIRS - 1)
        def _():
            pltpu.sync_copy(col_hbm.at[pl.ds(a + 3 * B, B)], colv1)
            pltpu.async_copy(xlin_hbm.at[colv1], rows1, sg1)
            wait_scatter(rowv1, wv1, ss1)
            pltpu.sync_copy(row_hbm.at[pl.ds(a + 3 * B, B)], rowv1)
            pltpu.async_copy(w_hbm.at[pl.ds(aw + 3 * B, B)], wv1, sw1)

    mult_scatter(colv0, rowv0, rows0, wv0, sg0, sw0, ss0)
    wait_scatter(rowv0, wv0, ss0)
    wait_scatter(rowv1, wv1, ss1)

    plsc.subcore_barrier()
    pltpu.sync_copy(agg.at[pl.ds(sid * ROWS_PER_SUBCORE, ROWS_PER_SUBCORE)],
                    out_hbm.at[cid, pl.ds(sid * ROWS_PER_SUBCORE,
                                          ROWS_PER_SUBCORE)])


@jax.jit
def kernel(x, edge_index, edge_attr, W1, b1, W2, b2, Wf1, bf1, Wf2, bf2):
    f32 = jnp.float32
    b1r = b1.reshape(1, D)
    b2r = b2.reshape(1, D)
    bf1r = bf1.reshape(1, D)
    bf2r = bf2.reshape(1, D)
    row = edge_index[0]
    col = edge_index[1]

    xlin = pl.pallas_call(
        _xlin_body,
        grid=(10,),
        in_specs=[
            pl.BlockSpec((N // 10, D), lambda i: (i, 0)),
            pl.BlockSpec((D, D), lambda i: (0, 0)),
            pl.BlockSpec((1, D), lambda i: (0, 0)),
        ],
        out_specs=pl.BlockSpec((N // 10, D), lambda i: (i, 0)),
        out_shape=jax.ShapeDtypeStruct((N, D), f32),
        compiler_params=pltpu.CompilerParams(
            dimension_semantics=("parallel",)),
    )(x, W1, b1r)

    BE = 4000
    sc_mesh = plsc.VectorSubcoreMesh(core_axis_name="core",
                                     subcore_axis_name="subcore")
    partials = []
    for c in range(NCHUNK):
        w_chunk = pl.pallas_call(
            _filter_body,
            grid=(EC // BE,),
            in_specs=[
                pl.BlockSpec((BE, NG), lambda i, c=c: (c * (EC // BE) + i, 0)),
                pl.BlockSpec((NG, D), lambda i: (0, 0)),
                pl.BlockSpec((1, D), lambda i: (0, 0)),
                pl.BlockSpec((D, D), lambda i: (0, 0)),
                pl.BlockSpec((1, D), lambda i: (0, 0)),
            ],
            out_specs=pl.BlockSpec((BE, D), lambda i: (i, 0)),
            out_shape=jax.ShapeDtypeStruct((EC, D), f32),
            compiler_params=pltpu.CompilerParams(
                dimension_semantics=("parallel",)),
        )(edge_attr, Wf1, bf1r, Wf2, bf2r)

        agg_c = pl.kernel(
            functools.partial(_sc_edge_body, c),
            out_type=jax.ShapeDtypeStruct((NUM_CORES, N_PAD, D), f32),
            mesh=sc_mesh,
            scratch_types=[
                pltpu.VMEM((B,), jnp.int32),
                pltpu.VMEM((B,), jnp.int32),
                pltpu.VMEM((B, D), f32),
                pltpu.VMEM((B, D), f32),
                pltpu.VMEM((B,), jnp.int32),
                pltpu.VMEM((B,), jnp.int32),
                pltpu.VMEM((B, D), f32),
                pltpu.VMEM((B, D), f32),
                pltpu.VMEM_SHARED((N_PAD, D), f32),
                pltpu.SemaphoreType.DMA,
                pltpu.SemaphoreType.DMA,
                pltpu.SemaphoreType.DMA,
                pltpu.SemaphoreType.DMA,
                pltpu.SemaphoreType.DMA,
                pltpu.SemaphoreType.DMA,
            ],
        )(xlin, col, row, w_chunk)
        partials.append(agg_c)

    out = pl.pallas_call(
        _out_body,
        grid=(10,),
        in_specs=[pl.BlockSpec((NUM_CORES, N // 10, D), lambda i: (0, i, 0))
                  for _ in range(NCHUNK)] + [
            pl.BlockSpec((D, D), lambda i: (0, 0)),
            pl.BlockSpec((1, D), lambda i: (0, 0)),
        ],
        out_specs=pl.BlockSpec((N // 10, D), lambda i: (i, 0)),
        out_shape=jax.ShapeDtypeStruct((N, D), f32),
        compiler_params=pltpu.CompilerParams(
            dimension_semantics=("parallel",)),
    )(*partials, W2, b2r)
    return out
